# Initial kernel scaffold; baseline (speedup 1.0000x reference)
import jax
import jax.numpy as jnp
from jax.experimental import pallas as pl


def kernel(x_nchw, conv1_w, conv1_b, conv2_w, conv2_b, fc1_w, fc1_b, fc2_w, fc2_b, fc3_w, fc3_b):
    raise NotImplementedError("write your pallas kernel here")


# speedup vs baseline: 48.3148x; 48.3148x over previous
import numpy as np
import jax
import jax.numpy as jnp
from jax.experimental import pallas as pl
from jax.experimental.pallas import tpu as pltpu

_BT = 64


def _conv1_mats(conv1_w, conv1_b):
    rl = np.arange(128)
    w_in, c_in = rl // 3, rl % 3
    cl = np.arange(256)
    half, rem = cl // 128, cl % 128
    j, oc = rem // 8, rem % 8
    ow = 2 * j + half
    DI = np.arange(5)[:, None, None]
    W = w_in[None, :, None]
    C = c_in[None, :, None]
    OW = ow[None, None, :]
    OC = np.broadcast_to(oc[None, None, :], (5, 128, 256))
    DJ = W - OW
    valid = (rl[None, :, None] < 96) & (OW < 28) & (DJ >= 0) & (DJ < 5)
    rows = (DI * 5 + np.clip(DJ, 0, 4)) * 3 + C
    rows = np.broadcast_to(rows, (5, 128, 256))
    a = jnp.where(valid, conv1_w[rows, OC], 0.0)
    a = a.reshape(640, 256).astype(jnp.bfloat16)
    brow = jnp.where(ow < 28, conv1_b[0, oc], 0.0).reshape(1, 256)
    return a, brow


def _conv2_mats(conv2_w, conv2_b):
    rl = np.arange(128)
    pw, cin = rl // 8, rl % 8
    cl = np.arange(256)
    half, rem = cl // 128, cl % 128
    j, oc = rem // 16, rem % 16
    ow2 = 2 * j + half
    DI = np.arange(5)[:, None, None]
    PW = pw[None, :, None]
    CI = cin[None, :, None]
    OW = ow2[None, None, :]
    OC = np.broadcast_to(oc[None, None, :], (5, 128, 256))
    DJ = PW - OW
    valid = (PW < 14) & (CI < 6) & (OW < 10) & (DJ >= 0) & (DJ < 5)
    rows = (DI * 5 + np.clip(DJ, 0, 4)) * 6 + np.clip(CI, 0, 5)
    rows = np.broadcast_to(rows, (5, 128, 256))
    b = jnp.where(valid, conv2_w[rows, OC], 0.0)
    b = b.reshape(640, 256).astype(jnp.bfloat16)
    brow = jnp.where(ow2 < 10, conv2_b[0, oc], 0.0).reshape(1, 256)
    return b, brow


def _lenet_body(x_ref, a_ref, ab_ref, b_ref, bb_ref, w1_ref, d1_ref,
                w2_ref, d2_ref, w3_ref, d3_ref, o_ref):
    bt = x_ref.shape[0]
    x = x_ref[...]
    x5 = jnp.concatenate([x[:, d:d + 32, :] for d in range(5)], axis=2)
    x5 = x5.reshape(bt * 32, 640)
    y = jnp.dot(x5, a_ref[...], preferred_element_type=jnp.float32)
    y = jnp.maximum(y + ab_ref[...], 0.0)
    yc = jnp.maximum(y[:, :128], y[:, 128:])
    yc = yc.reshape(bt, 16, 2, 128)
    p1 = jnp.maximum(yc[:, :, 0, :], yc[:, :, 1, :])
    p1 = p1.astype(jnp.bfloat16)
    p1 = jnp.concatenate(
        [p1, jnp.zeros((bt, 8, 128), jnp.bfloat16)], axis=1)
    x2 = jnp.concatenate([p1[:, d:d + 16, :] for d in range(5)], axis=2)
    x2 = x2.reshape(bt * 16, 640)
    y2 = jnp.dot(x2, b_ref[...], preferred_element_type=jnp.float32)
    y2 = jnp.maximum(y2 + bb_ref[...], 0.0)
    y2c = jnp.maximum(y2[:, :128], y2[:, 128:]).reshape(bt, 8, 2, 128)
    p2 = jnp.maximum(y2c[:, :, 0, :], y2c[:, :, 1, :])
    p2 = p2.astype(jnp.bfloat16)
    h = d1_ref[...]
    for r in range(5):
        h = h + jnp.dot(p2[:, r, :], w1_ref[r],
                        preferred_element_type=jnp.float32)
    h = jnp.maximum(h, 0.0).astype(jnp.bfloat16)
    h2 = jnp.dot(h, w2_ref[...], preferred_element_type=jnp.float32)
    h2 = jnp.maximum(h2 + d2_ref[...], 0.0).astype(jnp.bfloat16)
    out = jnp.dot(h2, w3_ref[...], preferred_element_type=jnp.float32)
    o_ref[...] = out + d3_ref[...]


def kernel(x_nchw, conv1_w, conv1_b, conv2_w, conv2_b,
           fc1_w, fc1_b, fc2_w, fc2_b, fc3_w, fc3_b):
    B = x_nchw.shape[0]
    bt = _BT if B % _BT == 0 else B
    x = jnp.transpose(x_nchw, (0, 2, 3, 1)).reshape(B, 32, 96)
    x = jnp.pad(x, ((0, 0), (0, 8), (0, 32))).astype(jnp.bfloat16)
    a_mat, a_bias = _conv1_mats(conv1_w, conv1_b)
    b_mat, b_bias = _conv2_mats(conv2_w, conv2_b)
    w1s = jnp.pad(fc1_w.reshape(5, 80, 128), ((0, 0), (0, 48), (0, 0)))

    grid = (B // bt,)
    m1, m2 = bt * 32, bt * 16
    cost = pl.CostEstimate(
        flops=(2 * m1 * 640 * 256 + 2 * m2 * 640 * 256
               + 7 * 2 * bt * 128 * 128) * grid[0],
        transcendentals=0,
        bytes_accessed=B * 40 * 128 * 2 + B * 128 * 4 + 4 * 640 * 256 * 2)
    out = pl.pallas_call(
        _lenet_body,
        out_shape=jax.ShapeDtypeStruct((B, 128), jnp.float32),
        grid=grid,
        in_specs=[
            pl.BlockSpec((bt, 40, 128), lambda i: (i, 0, 0)),
            pl.BlockSpec((640, 256), lambda i: (0, 0)),
            pl.BlockSpec((1, 256), lambda i: (0, 0)),
            pl.BlockSpec((640, 256), lambda i: (0, 0)),
            pl.BlockSpec((1, 256), lambda i: (0, 0)),
            pl.BlockSpec((5, 128, 128), lambda i: (0, 0, 0)),
            pl.BlockSpec((1, 128), lambda i: (0, 0)),
            pl.BlockSpec((128, 128), lambda i: (0, 0)),
            pl.BlockSpec((1, 128), lambda i: (0, 0)),
            pl.BlockSpec((128, 128), lambda i: (0, 0)),
            pl.BlockSpec((1, 128), lambda i: (0, 0)),
        ],
        out_specs=pl.BlockSpec((bt, 128), lambda i: (i, 0)),
        compiler_params=pltpu.CompilerParams(
            dimension_semantics=("parallel",),
            vmem_limit_bytes=100 * 1024 * 1024),
        cost_estimate=cost,
    )(x, a_mat, a_bias, b_mat, b_bias, w1s, fc1_b, fc2_w, fc2_b, fc3_w, fc3_b)
    return out[:, :10]


# speedup vs baseline: 48.4631x; 1.0031x over previous
import numpy as np
import jax
import jax.numpy as jnp
from jax.experimental import pallas as pl
from jax.experimental.pallas import tpu as pltpu

_BT = 64


def _conv1_mats(conv1_w, conv1_b):
    rl = np.arange(128)
    c_in, w_in = rl // 32, rl % 32
    cl = np.arange(256)
    half, rem = cl // 128, cl % 128
    j, oc = rem // 8, rem % 8
    ow = 2 * j + half
    DI = np.arange(5)[:, None, None]
    W = w_in[None, :, None]
    C = c_in[None, :, None]
    OW = ow[None, None, :]
    OC = np.broadcast_to(oc[None, None, :], (5, 128, 256))
    DJ = W - OW
    valid = (rl[None, :, None] < 96) & (OW < 28) & (DJ >= 0) & (DJ < 5)
    rows = (DI * 5 + np.clip(DJ, 0, 4)) * 3 + C
    rows = np.broadcast_to(rows, (5, 128, 256))
    a = jnp.where(valid, conv1_w[rows, OC], 0.0)
    a = a.reshape(640, 256).astype(jnp.bfloat16)
    brow = jnp.where(ow < 28, conv1_b[0, oc], 0.0).reshape(1, 256)
    return a, brow


def _conv2_mats(conv2_w, conv2_b):
    rl = np.arange(128)
    pw, cin = rl // 8, rl % 8
    cl = np.arange(256)
    half, rem = cl // 128, cl % 128
    j, oc = rem // 16, rem % 16
    ow2 = 2 * j + half
    DI = np.arange(5)[:, None, None]
    PW = pw[None, :, None]
    CI = cin[None, :, None]
    OW = ow2[None, None, :]
    OC = np.broadcast_to(oc[None, None, :], (5, 128, 256))
    DJ = PW - OW
    valid = (PW < 14) & (CI < 6) & (OW < 10) & (DJ >= 0) & (DJ < 5)
    rows = (DI * 5 + np.clip(DJ, 0, 4)) * 6 + np.clip(CI, 0, 5)
    rows = np.broadcast_to(rows, (5, 128, 256))
    b = jnp.where(valid, conv2_w[rows, OC], 0.0)
    b = b.reshape(640, 256).astype(jnp.bfloat16)
    brow = jnp.where(ow2 < 10, conv2_b[0, oc], 0.0).reshape(1, 256)
    return b, brow


def _lenet_body(x_ref, a_ref, ab_ref, b_ref, bb_ref, w1_ref, d1_ref,
                w2_ref, d2_ref, w3_ref, d3_ref, o_ref):
    bt = x_ref.shape[0]
    x = x_ref[...]
    x5 = jnp.concatenate([x[:, d:d + 32, :] for d in range(5)], axis=2)
    x5 = x5.reshape(bt * 32, 640)
    y = jnp.dot(x5, a_ref[...], preferred_element_type=jnp.float32)
    y = jnp.maximum(y + ab_ref[...], 0.0)
    yc = jnp.maximum(y[:, :128], y[:, 128:])
    yc = yc.reshape(bt, 16, 2, 128)
    p1 = jnp.maximum(yc[:, :, 0, :], yc[:, :, 1, :])
    p1 = p1.astype(jnp.bfloat16)
    p1 = jnp.concatenate(
        [p1, jnp.zeros((bt, 8, 128), jnp.bfloat16)], axis=1)
    x2 = jnp.concatenate([p1[:, d:d + 16, :] for d in range(5)], axis=2)
    x2 = x2.reshape(bt * 16, 640)
    y2 = jnp.dot(x2, b_ref[...], preferred_element_type=jnp.float32)
    y2 = jnp.maximum(y2 + bb_ref[...], 0.0)
    y2c = jnp.maximum(y2[:, :128], y2[:, 128:]).reshape(bt, 8, 2, 128)
    p2 = jnp.maximum(y2c[:, :, 0, :], y2c[:, :, 1, :])
    p2 = p2.astype(jnp.bfloat16)
    h = d1_ref[...]
    for r in range(5):
        h = h + jnp.dot(p2[:, r, :], w1_ref[r],
                        preferred_element_type=jnp.float32)
    h = jnp.maximum(h, 0.0).astype(jnp.bfloat16)
    h2 = jnp.dot(h, w2_ref[...], preferred_element_type=jnp.float32)
    h2 = jnp.maximum(h2 + d2_ref[...], 0.0).astype(jnp.bfloat16)
    out = jnp.dot(h2, w3_ref[...], preferred_element_type=jnp.float32)
    o_ref[...] = out + d3_ref[...]


def kernel(x_nchw, conv1_w, conv1_b, conv2_w, conv2_b,
           fc1_w, fc1_b, fc2_w, fc2_b, fc3_w, fc3_b):
    B = x_nchw.shape[0]
    bt = _BT if B % _BT == 0 else B
    x = jnp.transpose(x_nchw, (0, 2, 1, 3)).reshape(B, 32, 96)
    x = jnp.pad(x, ((0, 0), (0, 8), (0, 32))).astype(jnp.bfloat16)
    a_mat, a_bias = _conv1_mats(conv1_w, conv1_b)
    b_mat, b_bias = _conv2_mats(conv2_w, conv2_b)
    w1s = jnp.pad(fc1_w.reshape(5, 80, 128), ((0, 0), (0, 48), (0, 0)))

    grid = (B // bt,)
    m1, m2 = bt * 32, bt * 16
    cost = pl.CostEstimate(
        flops=(2 * m1 * 640 * 256 + 2 * m2 * 640 * 256
               + 7 * 2 * bt * 128 * 128) * grid[0],
        transcendentals=0,
        bytes_accessed=B * 40 * 128 * 2 + B * 128 * 4 + 4 * 640 * 256 * 2)
    out = pl.pallas_call(
        _lenet_body,
        out_shape=jax.ShapeDtypeStruct((B, 128), jnp.float32),
        grid=grid,
        in_specs=[
            pl.BlockSpec((bt, 40, 128), lambda i: (i, 0, 0)),
            pl.BlockSpec((640, 256), lambda i: (0, 0)),
            pl.BlockSpec((1, 256), lambda i: (0, 0)),
            pl.BlockSpec((640, 256), lambda i: (0, 0)),
            pl.BlockSpec((1, 256), lambda i: (0, 0)),
            pl.BlockSpec((5, 128, 128), lambda i: (0, 0, 0)),
            pl.BlockSpec((1, 128), lambda i: (0, 0)),
            pl.BlockSpec((128, 128), lambda i: (0, 0)),
            pl.BlockSpec((1, 128), lambda i: (0, 0)),
            pl.BlockSpec((128, 128), lambda i: (0, 0)),
            pl.BlockSpec((1, 128), lambda i: (0, 0)),
        ],
        out_specs=pl.BlockSpec((bt, 128), lambda i: (i, 0)),
        compiler_params=pltpu.CompilerParams(
            dimension_semantics=("parallel",),
            vmem_limit_bytes=100 * 1024 * 1024),
        cost_estimate=cost,
    )(x, a_mat, a_bias, b_mat, b_bias, w1s, fc1_b, fc2_w, fc2_b, fc3_w, fc3_b)
    return out[:, :10]


# speedup vs baseline: 914.8602x; 18.8775x over previous
import numpy as np
import jax
import jax.numpy as jnp
from jax.experimental import pallas as pl
from jax.experimental.pallas import tpu as pltpu

_BT = 64


def _conv1_mats(conv1_w, conv1_b):
    rl = np.arange(128)
    c_in, w_in = rl // 32, rl % 32
    hj = np.arange(32)
    half, j = hj // 16, hj % 16
    ow = 2 * j + half
    DI = np.arange(5)[:, None, None]
    DJ = w_in[None, :, None] - ow[None, None, :]
    valid = (rl[None, :, None] < 96) & (ow[None, None, :] < 28) \
        & (DJ >= 0) & (DJ < 5)
    krow = (DI * 5 + np.clip(DJ, 0, 4)) * 3 + c_in[None, :, None]
    krow = np.broadcast_to(np.clip(krow, 0, 74), (5, 128, 32))
    sel = np.zeros((5, 128, 32, 75), np.float32)
    np.put_along_axis(sel, krow[..., None],
                      valid[..., None].astype(np.float32), axis=-1)
    sel = jnp.asarray(sel.reshape(5 * 128 * 32, 75), jnp.bfloat16)
    a = jnp.dot(sel, conv1_w[:, :8], preferred_element_type=jnp.float32)
    a = a.reshape(640, 256).astype(jnp.bfloat16)
    cl = np.arange(256)
    ow_c = 2 * ((cl % 128) // 8) + cl // 128
    sel_b = np.zeros((256, 128), np.float32)
    sel_b[np.arange(256), cl % 8] = (ow_c < 28)
    brow = jnp.dot(conv1_b, jnp.asarray(sel_b.T))
    return a, brow


def _conv2_mats(conv2_w, conv2_b):
    rl = np.arange(128)
    pw, cin = rl // 8, rl % 8
    hj = np.arange(16)
    half, j = hj // 8, hj % 8
    ow2 = 2 * j + half
    DI = np.arange(5)[:, None, None]
    DJ = pw[None, :, None] - ow2[None, None, :]
    valid = (pw[None, :, None] < 14) & (cin[None, :, None] < 6) \
        & (ow2[None, None, :] < 10) & (DJ >= 0) & (DJ < 5)
    krow = (DI * 5 + np.clip(DJ, 0, 4)) * 6 + np.clip(cin, 0, 5)[None, :, None]
    krow = np.broadcast_to(krow, (5, 128, 16))
    sel = np.zeros((5, 128, 16, 150), np.float32)
    np.put_along_axis(sel, krow[..., None],
                      valid[..., None].astype(np.float32), axis=-1)
    sel = jnp.asarray(sel.reshape(5 * 128 * 16, 150), jnp.bfloat16)
    b = jnp.dot(sel, conv2_w[:, :16], preferred_element_type=jnp.float32)
    b = b.reshape(640, 256).astype(jnp.bfloat16)
    cl = np.arange(256)
    ow_c = 2 * ((cl % 128) // 16) + cl // 128
    sel_b = np.zeros((256, 128), np.float32)
    sel_b[np.arange(256), cl % 16] = (ow_c < 10)
    brow = jnp.dot(conv2_b, jnp.asarray(sel_b.T))
    return b, brow


def _lenet_body(x_ref, a_ref, ab_ref, b_ref, bb_ref, w1_ref, d1_ref,
                w2_ref, d2_ref, w3_ref, d3_ref, o_ref):
    bt = x_ref.shape[0]
    x = x_ref[...]
    x5 = jnp.concatenate([x[:, d:d + 32, :] for d in range(5)], axis=2)
    x5 = x5.reshape(bt * 32, 640)
    y = jnp.dot(x5, a_ref[...], preferred_element_type=jnp.float32)
    y = jnp.maximum(y + ab_ref[...], 0.0)
    yc = jnp.maximum(y[:, :128], y[:, 128:])
    yc = yc.reshape(bt, 16, 2, 128)
    p1 = jnp.maximum(yc[:, :, 0, :], yc[:, :, 1, :])
    p1 = p1.astype(jnp.bfloat16)
    p1 = jnp.concatenate(
        [p1, jnp.zeros((bt, 8, 128), jnp.bfloat16)], axis=1)
    x2 = jnp.concatenate([p1[:, d:d + 16, :] for d in range(5)], axis=2)
    x2 = x2.reshape(bt * 16, 640)
    y2 = jnp.dot(x2, b_ref[...], preferred_element_type=jnp.float32)
    y2 = jnp.maximum(y2 + bb_ref[...], 0.0)
    y2c = jnp.maximum(y2[:, :128], y2[:, 128:]).reshape(bt, 8, 2, 128)
    p2 = jnp.maximum(y2c[:, :, 0, :], y2c[:, :, 1, :])
    p2 = p2.astype(jnp.bfloat16)
    h = d1_ref[...]
    for r in range(5):
        h = h + jnp.dot(p2[:, r, :], w1_ref[r],
                        preferred_element_type=jnp.float32)
    h = jnp.maximum(h, 0.0).astype(jnp.bfloat16)
    h2 = jnp.dot(h, w2_ref[...], preferred_element_type=jnp.float32)
    h2 = jnp.maximum(h2 + d2_ref[...], 0.0).astype(jnp.bfloat16)
    out = jnp.dot(h2, w3_ref[...], preferred_element_type=jnp.float32)
    o_ref[...] = out + d3_ref[...]


def kernel(x_nchw, conv1_w, conv1_b, conv2_w, conv2_b,
           fc1_w, fc1_b, fc2_w, fc2_b, fc3_w, fc3_b):
    B = x_nchw.shape[0]
    bt = _BT if B % _BT == 0 else B
    x = jnp.transpose(x_nchw, (0, 2, 1, 3)).reshape(B, 32, 96)
    x = jnp.pad(x, ((0, 0), (0, 8), (0, 32))).astype(jnp.bfloat16)
    a_mat, a_bias = _conv1_mats(conv1_w, conv1_b)
    b_mat, b_bias = _conv2_mats(conv2_w, conv2_b)
    w1s = jnp.pad(fc1_w.reshape(5, 80, 128), ((0, 0), (0, 48), (0, 0)))

    grid = (B // bt,)
    m1, m2 = bt * 32, bt * 16
    cost = pl.CostEstimate(
        flops=(2 * m1 * 640 * 256 + 2 * m2 * 640 * 256
               + 7 * 2 * bt * 128 * 128) * grid[0],
        transcendentals=0,
        bytes_accessed=B * 40 * 128 * 2 + B * 128 * 4 + 4 * 640 * 256 * 2)
    out = pl.pallas_call(
        _lenet_body,
        out_shape=jax.ShapeDtypeStruct((B, 128), jnp.float32),
        grid=grid,
        in_specs=[
            pl.BlockSpec((bt, 40, 128), lambda i: (i, 0, 0)),
            pl.BlockSpec((640, 256), lambda i: (0, 0)),
            pl.BlockSpec((1, 256), lambda i: (0, 0)),
            pl.BlockSpec((640, 256), lambda i: (0, 0)),
            pl.BlockSpec((1, 256), lambda i: (0, 0)),
            pl.BlockSpec((5, 128, 128), lambda i: (0, 0, 0)),
            pl.BlockSpec((1, 128), lambda i: (0, 0)),
            pl.BlockSpec((128, 128), lambda i: (0, 0)),
            pl.BlockSpec((1, 128), lambda i: (0, 0)),
            pl.BlockSpec((128, 128), lambda i: (0, 0)),
            pl.BlockSpec((1, 128), lambda i: (0, 0)),
        ],
        out_specs=pl.BlockSpec((bt, 128), lambda i: (i, 0)),
        compiler_params=pltpu.CompilerParams(
            dimension_semantics=("parallel",),
            vmem_limit_bytes=100 * 1024 * 1024),
        cost_estimate=cost,
    )(x, a_mat, a_bias, b_mat, b_bias, w1s, fc1_b, fc2_w, fc2_b, fc3_w, fc3_b)
    return out[:, :10]


# speedup vs baseline: 954.2522x; 1.0431x over previous
import numpy as np
import jax
import jax.numpy as jnp
from jax.experimental import pallas as pl
from jax.experimental.pallas import tpu as pltpu

_BT = 128


def _conv1_mats(conv1_w, conv1_b):
    rl = np.arange(128)
    c_in, w_in = rl // 32, rl % 32
    hj = np.arange(32)
    half, j = hj // 16, hj % 16
    ow = 2 * j + half
    DI = np.arange(5)[:, None, None]
    DJ = w_in[None, :, None] - ow[None, None, :]
    valid = (rl[None, :, None] < 96) & (ow[None, None, :] < 28) \
        & (DJ >= 0) & (DJ < 5)
    krow = (DI * 5 + np.clip(DJ, 0, 4)) * 3 + c_in[None, :, None]
    krow = np.broadcast_to(np.clip(krow, 0, 74), (5, 128, 32))
    sel = np.zeros((5, 128, 32, 75), np.float32)
    np.put_along_axis(sel, krow[..., None],
                      valid[..., None].astype(np.float32), axis=-1)
    sel = jnp.asarray(sel.reshape(5 * 128 * 32, 75), jnp.bfloat16)
    a = jnp.dot(sel, conv1_w[:, :8], preferred_element_type=jnp.float32)
    a = a.reshape(640, 256).astype(jnp.bfloat16)
    cl = np.arange(256)
    ow_c = 2 * ((cl % 128) // 8) + cl // 128
    sel_b = np.zeros((256, 128), np.float32)
    sel_b[np.arange(256), cl % 8] = (ow_c < 28)
    brow = jnp.dot(conv1_b, jnp.asarray(sel_b.T))
    return a, brow


def _conv2_mats(conv2_w, conv2_b):
    rl = np.arange(128)
    pw, cin = rl // 8, rl % 8
    hj = np.arange(16)
    half, j = hj // 8, hj % 8
    ow2 = 2 * j + half
    DI = np.arange(5)[:, None, None]
    DJ = pw[None, :, None] - ow2[None, None, :]
    valid = (pw[None, :, None] < 14) & (cin[None, :, None] < 6) \
        & (ow2[None, None, :] < 10) & (DJ >= 0) & (DJ < 5)
    krow = (DI * 5 + np.clip(DJ, 0, 4)) * 6 + np.clip(cin, 0, 5)[None, :, None]
    krow = np.broadcast_to(krow, (5, 128, 16))
    sel = np.zeros((5, 128, 16, 150), np.float32)
    np.put_along_axis(sel, krow[..., None],
                      valid[..., None].astype(np.float32), axis=-1)
    sel = jnp.asarray(sel.reshape(5 * 128 * 16, 150), jnp.bfloat16)
    b = jnp.dot(sel, conv2_w[:, :16], preferred_element_type=jnp.float32)
    b = b.reshape(640, 256).astype(jnp.bfloat16)
    cl = np.arange(256)
    ow_c = 2 * ((cl % 128) // 16) + cl // 128
    sel_b = np.zeros((256, 128), np.float32)
    sel_b[np.arange(256), cl % 16] = (ow_c < 10)
    brow = jnp.dot(conv2_b, jnp.asarray(sel_b.T))
    return b, brow


def _lenet_body(x_ref, a_ref, ab_ref, b_ref, bb_ref, w1_ref, d1_ref,
                w2_ref, d2_ref, w3_ref, d3_ref, o_ref):
    bt = x_ref.shape[0]
    x = x_ref[...]
    x5 = jnp.concatenate([x[:, d:d + 32, :] for d in range(5)], axis=2)
    x5 = x5.reshape(bt * 32, 640)
    y = jnp.dot(x5, a_ref[...], preferred_element_type=jnp.float32)
    y = jnp.maximum(y + ab_ref[...], 0.0)
    yc = jnp.maximum(y[:, :128], y[:, 128:])
    yc = yc.reshape(bt, 16, 2, 128)
    p1 = jnp.maximum(yc[:, :, 0, :], yc[:, :, 1, :])
    p1 = p1.astype(jnp.bfloat16)
    p1 = jnp.concatenate(
        [p1, jnp.zeros((bt, 8, 128), jnp.bfloat16)], axis=1)
    x2 = jnp.concatenate([p1[:, d:d + 16, :] for d in range(5)], axis=2)
    x2 = x2.reshape(bt * 16, 640)
    y2 = jnp.dot(x2, b_ref[...], preferred_element_type=jnp.float32)
    y2 = jnp.maximum(y2 + bb_ref[...], 0.0)
    y2c = jnp.maximum(y2[:, :128], y2[:, 128:]).reshape(bt, 8, 2, 128)
    p2 = jnp.maximum(y2c[:, :, 0, :], y2c[:, :, 1, :])
    p2 = p2.astype(jnp.bfloat16)
    h = d1_ref[...]
    for r in range(5):
        h = h + jnp.dot(p2[:, r, :], w1_ref[r],
                        preferred_element_type=jnp.float32)
    h = jnp.maximum(h, 0.0).astype(jnp.bfloat16)
    h2 = jnp.dot(h, w2_ref[...], preferred_element_type=jnp.float32)
    h2 = jnp.maximum(h2 + d2_ref[...], 0.0).astype(jnp.bfloat16)
    out = jnp.dot(h2, w3_ref[...], preferred_element_type=jnp.float32)
    o_ref[...] = out + d3_ref[...]


def kernel(x_nchw, conv1_w, conv1_b, conv2_w, conv2_b,
           fc1_w, fc1_b, fc2_w, fc2_b, fc3_w, fc3_b):
    B = x_nchw.shape[0]
    bt = _BT if B % _BT == 0 else B
    x = jnp.transpose(x_nchw, (0, 2, 1, 3)).reshape(B, 32, 96)
    x = jnp.pad(x, ((0, 0), (0, 8), (0, 32))).astype(jnp.bfloat16)
    a_mat, a_bias = _conv1_mats(conv1_w, conv1_b)
    b_mat, b_bias = _conv2_mats(conv2_w, conv2_b)
    w1s = jnp.pad(fc1_w.reshape(5, 80, 128), ((0, 0), (0, 48), (0, 0)))

    grid = (B // bt,)
    m1, m2 = bt * 32, bt * 16
    cost = pl.CostEstimate(
        flops=(2 * m1 * 640 * 256 + 2 * m2 * 640 * 256
               + 7 * 2 * bt * 128 * 128) * grid[0],
        transcendentals=0,
        bytes_accessed=B * 40 * 128 * 2 + B * 128 * 4 + 4 * 640 * 256 * 2)
    out = pl.pallas_call(
        _lenet_body,
        out_shape=jax.ShapeDtypeStruct((B, 128), jnp.float32),
        grid=grid,
        in_specs=[
            pl.BlockSpec((bt, 40, 128), lambda i: (i, 0, 0)),
            pl.BlockSpec((640, 256), lambda i: (0, 0)),
            pl.BlockSpec((1, 256), lambda i: (0, 0)),
            pl.BlockSpec((640, 256), lambda i: (0, 0)),
            pl.BlockSpec((1, 256), lambda i: (0, 0)),
            pl.BlockSpec((5, 128, 128), lambda i: (0, 0, 0)),
            pl.BlockSpec((1, 128), lambda i: (0, 0)),
            pl.BlockSpec((128, 128), lambda i: (0, 0)),
            pl.BlockSpec((1, 128), lambda i: (0, 0)),
            pl.BlockSpec((128, 128), lambda i: (0, 0)),
            pl.BlockSpec((1, 128), lambda i: (0, 0)),
        ],
        out_specs=pl.BlockSpec((bt, 128), lambda i: (i, 0)),
        compiler_params=pltpu.CompilerParams(
            dimension_semantics=("parallel",),
            vmem_limit_bytes=100 * 1024 * 1024),
        cost_estimate=cost,
    )(x, a_mat, a_bias, b_mat, b_bias, w1s, fc1_b, fc2_w, fc2_b, fc3_w, fc3_b)
    return out[:, :10]
